# baseline (speedup 1.0000x reference)
import functools

import jax
import jax.numpy as jnp
from jax.experimental import pallas as pl
from jax.experimental.pallas import tpu as pltpu

BN_EPS = 1e-5
LEAKY_SLOPE = 0.01

KB1 = 512
BM = 256


def _stats_kernel(x_ref, gamma_ref, beta_ref, s_ref, t_ref):
    x = x_ref[...]
    n = x.shape[0]
    mean = jnp.sum(x, axis=0, keepdims=True) * (1.0 / n)
    ex2 = jnp.sum(x * x, axis=0, keepdims=True) * (1.0 / n)
    var = ex2 - mean * mean
    s = gamma_ref[...] * jax.lax.rsqrt(var + BN_EPS)
    s_ref[...] = s
    t_ref[...] = beta_ref[...] - mean * s


def _mm_kernel(x_ref, s_ref, t_ref, w_ref, b_ref, o_ref):
    h = (x_ref[...] * s_ref[...] + t_ref[...]).astype(jnp.bfloat16)
    acc = jax.lax.dot_general(
        h, w_ref[...],
        dimension_numbers=(((1,), (0,)), ((), ())),
        preferred_element_type=jnp.float32,
    )
    y = acc + b_ref[...]
    o_ref[...] = jnp.where(y >= 0.0, y, LEAKY_SLOPE * y)


@functools.partial(jax.jit, donate_argnums=())
def kernel(x, gamma, beta, W, b):
    B, IN = x.shape
    OUT = W.shape[0]

    gamma2 = gamma.reshape(1, IN)
    beta2 = beta.reshape(1, IN)
    b2 = b.reshape(1, OUT)
    Wt16 = W.T.astype(jnp.bfloat16)

    s, t = pl.pallas_call(
        _stats_kernel,
        grid=(IN // KB1,),
        in_specs=[
            pl.BlockSpec((B, KB1), lambda k: (0, k)),
            pl.BlockSpec((1, KB1), lambda k: (0, k)),
            pl.BlockSpec((1, KB1), lambda k: (0, k)),
        ],
        out_specs=[
            pl.BlockSpec((1, KB1), lambda k: (0, k)),
            pl.BlockSpec((1, KB1), lambda k: (0, k)),
        ],
        out_shape=[
            jax.ShapeDtypeStruct((1, IN), jnp.float32),
            jax.ShapeDtypeStruct((1, IN), jnp.float32),
        ],
        compiler_params=pltpu.CompilerParams(
            dimension_semantics=("arbitrary",),
        ),
    )(x, gamma2, beta2)

    out = pl.pallas_call(
        _mm_kernel,
        grid=(B // BM,),
        in_specs=[
            pl.BlockSpec((BM, IN), lambda m: (m, 0)),
            pl.BlockSpec((1, IN), lambda m: (0, 0)),
            pl.BlockSpec((1, IN), lambda m: (0, 0)),
            pl.BlockSpec((IN, OUT), lambda m: (0, 0)),
            pl.BlockSpec((1, OUT), lambda m: (0, 0)),
        ],
        out_specs=pl.BlockSpec((BM, OUT), lambda m: (m, 0)),
        out_shape=jax.ShapeDtypeStruct((B, OUT), jnp.float32),
        compiler_params=pltpu.CompilerParams(
            dimension_semantics=("arbitrary",),
            vmem_limit_bytes=62 * 1024 * 1024,
        ),
    )(x, s, t, Wt16, b2)
    return out


# speedup vs baseline: 1.0456x; 1.0456x over previous
import functools

import jax
import jax.numpy as jnp
from jax.experimental import pallas as pl
from jax.experimental.pallas import tpu as pltpu

BN_EPS = 1e-5
LEAKY_SLOPE = 0.01

KB1 = 512
BM = 512
BN = 2048


def _stats_kernel(x_ref, gamma_ref, beta_ref, s_ref, t_ref):
    x = x_ref[...]
    n = x.shape[0]
    mean = jnp.sum(x, axis=0, keepdims=True) * (1.0 / n)
    ex2 = jnp.sum(x * x, axis=0, keepdims=True) * (1.0 / n)
    var = ex2 - mean * mean
    s = gamma_ref[...] * jax.lax.rsqrt(var + BN_EPS)
    s_ref[...] = s
    t_ref[...] = beta_ref[...] - mean * s


def _mm_kernel(x_ref, s_ref, t_ref, w_ref, b_ref, o_ref):
    h = (x_ref[...] * s_ref[...] + t_ref[...]).astype(jnp.bfloat16)
    acc = jax.lax.dot_general(
        h, w_ref[...],
        dimension_numbers=(((1,), (1,)), ((), ())),
        preferred_element_type=jnp.float32,
    )
    y = acc + b_ref[...]
    o_ref[...] = jnp.where(y >= 0.0, y, LEAKY_SLOPE * y)


@functools.partial(jax.jit, donate_argnums=())
def kernel(x, gamma, beta, W, b):
    B, IN = x.shape
    OUT = W.shape[0]

    gamma2 = gamma.reshape(1, IN)
    beta2 = beta.reshape(1, IN)
    b2 = b.reshape(1, OUT)
    W16 = W.astype(jnp.bfloat16)

    s, t = pl.pallas_call(
        _stats_kernel,
        grid=(IN // KB1,),
        in_specs=[
            pl.BlockSpec((B, KB1), lambda k: (0, k)),
            pl.BlockSpec((1, KB1), lambda k: (0, k)),
            pl.BlockSpec((1, KB1), lambda k: (0, k)),
        ],
        out_specs=[
            pl.BlockSpec((1, KB1), lambda k: (0, k)),
            pl.BlockSpec((1, KB1), lambda k: (0, k)),
        ],
        out_shape=[
            jax.ShapeDtypeStruct((1, IN), jnp.float32),
            jax.ShapeDtypeStruct((1, IN), jnp.float32),
        ],
        compiler_params=pltpu.CompilerParams(
            dimension_semantics=("arbitrary",),
        ),
    )(x, gamma2, beta2)

    out = pl.pallas_call(
        _mm_kernel,
        grid=(OUT // BN, B // BM),
        in_specs=[
            pl.BlockSpec((BM, IN), lambda n, m: (m, 0)),
            pl.BlockSpec((1, IN), lambda n, m: (0, 0)),
            pl.BlockSpec((1, IN), lambda n, m: (0, 0)),
            pl.BlockSpec((BN, IN), lambda n, m: (n, 0)),
            pl.BlockSpec((1, BN), lambda n, m: (0, n)),
        ],
        out_specs=pl.BlockSpec((BM, BN), lambda n, m: (m, n)),
        out_shape=jax.ShapeDtypeStruct((B, OUT), jnp.float32),
        compiler_params=pltpu.CompilerParams(
            dimension_semantics=("arbitrary", "arbitrary"),
            vmem_limit_bytes=62 * 1024 * 1024,
        ),
    )(x, s, t, W16, b2)
    return out
